# baseline (speedup 1.0000x reference)
import functools

import jax
import jax.numpy as jnp
from jax import lax
from jax.experimental import pallas as pl
from jax.experimental.pallas import tpu as pltpu
from jax.experimental.pallas import tpu_sc as plsc

EMBED_DIM = 64
NC = 2
NS = 16
NW = NC * NS
SUB = 128
CHUNK = 256
NHALF = CHUNK // SUB
NIN = 4
NBUF = 2


def _sc_body(ct_hbm, pt_hbm, ci_hbm, pi_hbm, oc_hbm, loc_hbm, w_hbm,
             out_hbm,
             ci_v, pi_v, oc_v, pi2_v, loc_v, rc_v, rp_v, o_v, w_v,
             sem_a, sem_g, sem_s, n_tokens):
    per_w = n_tokens // NW
    n_chunks = per_w // CHUNK
    wid = lax.axis_index("s") * NC + lax.axis_index("c")
    base = wid * per_w
    rbase = base // SUB

    pltpu.sync_copy(w_hbm, w_v)
    w0 = [w_v[0, pl.ds(16 * j, 16)] for j in range(4)]
    w1 = [w_v[1, pl.ds(16 * j, 16)] for j in range(4)]

    def fire_a(ck, slot):
        roff = rbase + ck * NHALF
        loff = 2 * (base + ck * CHUNK)
        pltpu.async_copy(ci_hbm.at[pl.ds(roff, NHALF)], ci_v.at[slot], sem_a[slot])
        pltpu.async_copy(pi_hbm.at[pl.ds(roff, NHALF)], pi_v.at[slot], sem_a[slot])
        pltpu.async_copy(oc_hbm.at[pl.ds(roff, NHALF)], oc_v.at[slot], sem_a[slot])
        pltpu.async_copy(loc_hbm.at[pl.ds(loff, 2 * CHUNK)], loc_v.at[slot], sem_a[slot])

    def wait_a(slot):
        pltpu.make_async_copy(ci_hbm.at[pl.ds(0, NHALF)], ci_v.at[slot], sem_a[slot]).wait()
        pltpu.make_async_copy(pi_hbm.at[pl.ds(0, NHALF)], pi_v.at[slot], sem_a[slot]).wait()
        pltpu.make_async_copy(oc_hbm.at[pl.ds(0, NHALF)], oc_v.at[slot], sem_a[slot]).wait()
        pltpu.make_async_copy(loc_hbm.at[pl.ds(0, 2 * CHUNK)], loc_v.at[slot], sem_a[slot]).wait()

    def compute_idx(slot):
        for h in range(NHALF):
            for m in range(SUB // 16):
                s = pl.ds(16 * m, 16)
                pi2_v[slot, h, s] = pi_v[slot, h, s] * 2 + oc_v[slot, h, s]

    def fire_g(slot, p):
        for h in range(NHALF):
            pltpu.async_copy(ct_hbm.at[ci_v.at[slot, h]],
                             rc_v.at[p, pl.ds(SUB * h, SUB)], sem_g[p])
            pltpu.async_copy(pt_hbm.at[pi2_v.at[slot, h]],
                             rp_v.at[p, pl.ds(SUB * h, SUB)], sem_g[p])

    def wait_g(slot, p):
        for h in range(NHALF):
            pltpu.make_async_copy(ct_hbm.at[ci_v.at[slot, h]],
                                  rc_v.at[p, pl.ds(SUB * h, SUB)], sem_g[p]).wait()
            pltpu.make_async_copy(pt_hbm.at[pi2_v.at[slot, h]],
                                  rp_v.at[p, pl.ds(SUB * h, SUB)], sem_g[p]).wait()

    def fire_s(ck, p):
        off = base + ck * CHUNK
        pltpu.async_copy(o_v.at[p], out_hbm.at[pl.ds(off, CHUNK)], sem_s[p])

    def wait_s(p):
        pltpu.make_async_copy(o_v.at[p], out_hbm.at[pl.ds(0, CHUNK)], sem_s[p]).wait()

    def compute(slot, p):
        def grp_body(g, tc):
            lv = loc_v[slot, pl.ds(16 * g, 16)]
            for i in range(8):
                xs = lv[2 * i]
                ys = lv[2 * i + 1]
                t = 8 * g + i
                for j in range(4):
                    s = pl.ds(16 * j, 16)
                    o_v[p, t, s] = ((rc_v[p, t, s] + rp_v[p, t, s])
                                    + (xs * w0[j] + ys * w1[j]))
            return tc

        lax.fori_loop(0, CHUNK // 8, grp_body, 0)

    for u in range(NIN):
        fire_a(u, u)
    wait_a(0)
    compute_idx(0)
    fire_g(0, 0)

    def quad_body(g2, carry):
        for u in range(NIN):
            ck = g2 * NIN + u
            slot = u
            p = u & 1
            q = 1 - p
            nslot = (u + 1) % NIN

            @pl.when(ck + 1 < n_chunks)
            def _():
                wait_a(nslot)
                compute_idx(nslot)
                fire_g(nslot, q)

            wait_g(slot, p)

            @pl.when(ck >= NBUF)
            def _():
                wait_s(p)

            compute(slot, p)
            fire_s(ck, p)

            @pl.when(ck + NIN < n_chunks)
            def _():
                fire_a(ck + NIN, slot)
        return carry

    lax.fori_loop(0, n_chunks // NIN, quad_body, 0)
    wait_s(0)
    wait_s(1)


def kernel(contour_tensor, point_tensor, location_tensor, on_curve_tensor,
           contour_table, point_table, oncurve_table, loc_W, loc_b):
    B, L = contour_tensor.shape
    n = B * L
    scale = float(EMBED_DIM) ** 0.5

    ct2 = contour_table[1:] * scale
    pt2 = ((point_table[1:, None, :] + oncurve_table[None, 1:3, :]
            + loc_b[None, None, :]) * scale).reshape(-1, EMBED_DIM)
    w8 = loc_W.T * scale

    ci = contour_tensor.reshape(n // SUB, SUB).astype(jnp.int32)
    pi = point_tensor.reshape(n // SUB, SUB).astype(jnp.int32)
    oc = on_curve_tensor.reshape(n // SUB, SUB).astype(jnp.int32)
    loc = location_tensor.reshape(2 * n)

    mesh = plsc.VectorSubcoreMesh(core_axis_name="c", subcore_axis_name="s")
    run = functools.partial(
        pl.kernel,
        mesh=mesh,
        compiler_params=pltpu.CompilerParams(use_tc_tiling_on_sc=False),
        out_type=jax.ShapeDtypeStruct((n, EMBED_DIM), jnp.float32),
        scratch_types=[
            pltpu.VMEM((NIN, NHALF, SUB), jnp.int32),
            pltpu.VMEM((NIN, NHALF, SUB), jnp.int32),
            pltpu.VMEM((NIN, NHALF, SUB), jnp.int32),
            pltpu.VMEM((NIN, NHALF, SUB), jnp.int32),
            pltpu.VMEM((NIN, 2 * CHUNK), jnp.float32),
            pltpu.VMEM((NBUF, CHUNK, EMBED_DIM), jnp.float32),
            pltpu.VMEM((NBUF, CHUNK, EMBED_DIM), jnp.float32),
            pltpu.VMEM((NBUF, CHUNK, EMBED_DIM), jnp.float32),
            pltpu.VMEM((2, EMBED_DIM), jnp.float32),
            [pltpu.SemaphoreType.DMA] * NIN,
            [pltpu.SemaphoreType.DMA] * NBUF,
            [pltpu.SemaphoreType.DMA] * NBUF,
        ],
    )(functools.partial(_sc_body, n_tokens=n))
    out = run(ct2, pt2, ci, pi, oc, loc, w8)
    return out.reshape(B, L, EMBED_DIM)


# speedup vs baseline: 1.8016x; 1.8016x over previous
import functools

import jax
import jax.numpy as jnp
from jax import lax
from jax.experimental import pallas as pl
from jax.experimental.pallas import tpu as pltpu
from jax.experimental.pallas import tpu_sc as plsc

EMBED_DIM = 64
NC = 2
NS = 16
NW = NC * NS
NIN = 4
NBUF = 2


def _sc_body(ct_hbm, pt_hbm, ci_hbm, pi_hbm, oc_hbm, x_hbm, y_hbm, w_hbm,
             out_hbm,
             ci_v, pi_v, oc_v, pi2_v, x_v, y_v, rc_v, rp_v, o_v, w_v,
             sem_a, sem_g, sem_s):
    B, L = ci_hbm.shape
    rows_w = B // NW
    pad = 8 * ((L + 15) // 16 * 16 // 8)
    splits = [(0, 104), (104, L - 104)]
    wid = lax.axis_index("s") * NC + lax.axis_index("c")
    base = wid * rows_w

    pltpu.sync_copy(w_hbm, w_v)
    w0 = [w_v[0, pl.ds(16 * j, 16)] for j in range(4)]
    w1 = [w_v[1, pl.ds(16 * j, 16)] for j in range(4)]

    def fire_a(ck, slot):
        b = base + ck
        pltpu.async_copy(ci_hbm.at[b], ci_v.at[slot, pl.ds(0, L)], sem_a[slot])
        pltpu.async_copy(pi_hbm.at[b], pi_v.at[slot, pl.ds(0, L)], sem_a[slot])
        pltpu.async_copy(oc_hbm.at[b], oc_v.at[slot, pl.ds(0, L)], sem_a[slot])
        pltpu.async_copy(x_hbm.at[b], x_v.at[slot, pl.ds(0, L)], sem_a[slot])
        pltpu.async_copy(y_hbm.at[b], y_v.at[slot, pl.ds(0, L)], sem_a[slot])

    def wait_a(slot):
        pltpu.make_async_copy(ci_hbm.at[0], ci_v.at[slot, pl.ds(0, L)], sem_a[slot]).wait()
        pltpu.make_async_copy(pi_hbm.at[0], pi_v.at[slot, pl.ds(0, L)], sem_a[slot]).wait()
        pltpu.make_async_copy(oc_hbm.at[0], oc_v.at[slot, pl.ds(0, L)], sem_a[slot]).wait()
        pltpu.make_async_copy(x_hbm.at[0], x_v.at[slot, pl.ds(0, L)], sem_a[slot]).wait()
        pltpu.make_async_copy(y_hbm.at[0], y_v.at[slot, pl.ds(0, L)], sem_a[slot]).wait()

    def compute_idx(slot):
        for m in range(pad // 16):
            s = pl.ds(16 * m, 16)
            pi2_v[slot, s] = pi_v[slot, s] * 2 + oc_v[slot, s]

    def fire_g(slot, p):
        for o, sz in splits:
            pltpu.async_copy(ct_hbm.at[ci_v.at[slot, pl.ds(o, sz)]],
                             rc_v.at[p, pl.ds(o, sz)], sem_g[p])
            pltpu.async_copy(pt_hbm.at[pi2_v.at[slot, pl.ds(o, sz)]],
                             rp_v.at[p, pl.ds(o, sz)], sem_g[p])

    def wait_g(slot, p):
        for o, sz in splits:
            pltpu.make_async_copy(ct_hbm.at[ci_v.at[slot, pl.ds(o, sz)]],
                                  rc_v.at[p, pl.ds(o, sz)], sem_g[p]).wait()
            pltpu.make_async_copy(pt_hbm.at[pi2_v.at[slot, pl.ds(o, sz)]],
                                  rp_v.at[p, pl.ds(o, sz)], sem_g[p]).wait()

    def fire_s(ck, p):
        pltpu.async_copy(o_v.at[p, pl.ds(0, L)], out_hbm.at[base + ck], sem_s[p])

    def wait_s(p):
        pltpu.make_async_copy(o_v.at[p, pl.ds(0, L)], out_hbm.at[0], sem_s[p]).wait()

    def compute(slot, p):
        def grp_body(g, tc):
            sg = pl.ds(16 * g, 16)
            xg = x_v[slot, sg]
            yg = y_v[slot, sg]
            for i in range(16):
                xs = xg[i]
                ys = yg[i]
                t = 16 * g + i
                for j in range(4):
                    s = pl.ds(16 * j, 16)
                    o_v[p, t, s] = ((rc_v[p, t, s] + rp_v[p, t, s])
                                    + (xs * w0[j] + ys * w1[j]))
            return tc

        lax.fori_loop(0, pad // 16, grp_body, 0)

    for u in range(NIN):
        fire_a(u, u)
    wait_a(0)
    compute_idx(0)
    fire_g(0, 0)

    def quad_body(g2, carry):
        for u in range(NIN):
            ck = g2 * NIN + u
            slot = u
            p = u & 1
            q = 1 - p
            nslot = (u + 1) % NIN

            @pl.when(ck + 1 < rows_w)
            def _():
                wait_a(nslot)
                compute_idx(nslot)
                fire_g(nslot, q)

            wait_g(slot, p)

            @pl.when(ck >= NBUF)
            def _():
                wait_s(p)

            compute(slot, p)
            fire_s(ck, p)

            @pl.when(ck + NIN < rows_w)
            def _():
                fire_a(ck + NIN, slot)
        return carry

    lax.fori_loop(0, rows_w // NIN, quad_body, 0)
    wait_s(0)
    wait_s(1)


def kernel(contour_tensor, point_tensor, location_tensor, on_curve_tensor,
           contour_table, point_table, oncurve_table, loc_W, loc_b):
    B, L = contour_tensor.shape
    scale = float(EMBED_DIM) ** 0.5
    pad = (L + 15) // 16 * 16

    ct2 = contour_table[1:] * scale
    pt2 = ((point_table[1:, None, :] + oncurve_table[None, 1:3, :]
            + loc_b[None, None, :]) * scale).reshape(-1, EMBED_DIM)
    w8 = loc_W.T * scale

    ci = contour_tensor.astype(jnp.int32)
    pi = point_tensor.astype(jnp.int32)
    oc = on_curve_tensor.astype(jnp.int32)
    x = location_tensor[..., 0]
    y = location_tensor[..., 1]

    mesh = plsc.VectorSubcoreMesh(core_axis_name="c", subcore_axis_name="s")
    run = functools.partial(
        pl.kernel,
        mesh=mesh,
        compiler_params=pltpu.CompilerParams(use_tc_tiling_on_sc=False),
        out_type=jax.ShapeDtypeStruct((B, L, EMBED_DIM), jnp.float32),
        scratch_types=[
            pltpu.VMEM((NIN, pad), jnp.int32),
            pltpu.VMEM((NIN, pad), jnp.int32),
            pltpu.VMEM((NIN, pad), jnp.int32),
            pltpu.VMEM((NIN, pad), jnp.int32),
            pltpu.VMEM((NIN, pad), jnp.float32),
            pltpu.VMEM((NIN, pad), jnp.float32),
            pltpu.VMEM((NBUF, pad, EMBED_DIM), jnp.float32),
            pltpu.VMEM((NBUF, pad, EMBED_DIM), jnp.float32),
            pltpu.VMEM((NBUF, pad, EMBED_DIM), jnp.float32),
            pltpu.VMEM((2, EMBED_DIM), jnp.float32),
            [pltpu.SemaphoreType.DMA] * NIN,
            [pltpu.SemaphoreType.DMA] * NBUF,
            [pltpu.SemaphoreType.DMA] * NBUF,
        ],
    )(_sc_body)
    return run(ct2, pt2, ci, pi, oc, x, y, w8)
